# baseline (speedup 1.0000x reference)
import jax
import jax.numpy as jnp
from jax import lax
from jax.experimental import pallas as pl
from jax.experimental.pallas import tpu as pltpu

WORDLEN = 100000
EMB = 64
CTX = 20
HID = 128
BK = 2048
G = 7
NJ = 7
PAD = G * NJ * BK
NEG = -jnp.inf


def _hkern(x_ref, *refs):
    tbl = refs[:CTX]
    w1_ref, b1_ref, h_out = refs[CTX], refs[CTX + 1], refs[CTX + 2]
    acc = b1_ref[...]
    for i in range(CTX):
        blk = tbl[i][...]
        sub = lax.rem(x_ref[i], 8)
        msk = lax.broadcasted_iota(jnp.int32, (8, EMB), 0) == sub
        row = jnp.sum(jnp.where(msk, blk, 0.0), axis=0, keepdims=True)
        acc = acc + jnp.dot(row, w1_ref[i * EMB:(i + 1) * EMB, :],
                            preferred_element_type=jnp.float32)
    h_out[...] = jnp.maximum(acc, 0.0)


def _run_h(x, table, W1, b1):
    b1r = b1.reshape(1, HID)
    tbl_specs = [
        pl.BlockSpec((8, EMB), lambda j, xr, i=i: (xr[i] // 8, 0))
        for i in range(CTX)
    ]
    grid_spec = pltpu.PrefetchScalarGridSpec(
        num_scalar_prefetch=1,
        grid=(1,),
        in_specs=[
            *tbl_specs,
            pl.BlockSpec((HID * 10, HID), lambda j, xr: (0, 0)),
            pl.BlockSpec((1, HID), lambda j, xr: (0, 0)),
        ],
        out_specs=pl.BlockSpec((1, HID), lambda j, xr: (0, 0)),
    )
    return pl.pallas_call(
        _hkern,
        grid_spec=grid_spec,
        out_shape=jax.ShapeDtypeStruct((1, HID), jnp.float32),
    )(x, *([table] * CTX), W1, b1r)


def _stream(h_ref, *rest):
    w2_blks = rest[:G]
    b2_ref, out_ref, m_ref = rest[G:]
    j = pl.program_id(0)

    @pl.when(j == 0)
    def _():
        m_ref[...] = jnp.full((1, BK), NEG, jnp.float32)

    h = h_ref[...]
    m = m_ref[...]
    for g in range(G):
        bidx = g * NJ + j
        logits = jnp.dot(h, w2_blks[g][...],
                         preferred_element_type=jnp.float32)
        logits = logits + b2_ref[:, pl.ds(bidx * BK, BK)]
        col = lax.broadcasted_iota(jnp.int32, (1, BK), 1) + bidx * BK
        logits = jnp.where(col < WORDLEN, logits, NEG)
        out_ref[:, pl.ds(bidx * BK, BK)] = logits
        m = jnp.maximum(m, logits)
    m_ref[...] = m

    @pl.when(j == NJ - 1)
    def _finalize():
        mx = jnp.max(m_ref[...])
        lo = out_ref[...]
        s = jnp.sum(jnp.exp(lo - mx))
        out_ref[...] = lo - (mx + jnp.log(s))


def kernel(x, table, W1, b1, W2, b2):
    h = _run_h(x, table, W1, b1)
    b2p = jnp.pad(b2, (0, PAD - WORDLEN)).reshape(1, PAD)

    w2_specs = [
        pl.BlockSpec((HID, BK), lambda j, g=g: (0, g * NJ + j))
        for g in range(G)
    ]
    out = pl.pallas_call(
        _stream,
        grid=(NJ,),
        in_specs=[
            pl.BlockSpec((1, HID), lambda j: (0, 0)),
            *w2_specs,
            pl.BlockSpec((1, PAD), lambda j: (0, 0)),
        ],
        out_specs=pl.BlockSpec((1, PAD), lambda j: (0, 0)),
        out_shape=jax.ShapeDtypeStruct((1, PAD), jnp.float32),
        scratch_shapes=[pltpu.VMEM((1, BK), jnp.float32)],
    )(h, *([W2] * G), b2p)
    return out[:, :WORDLEN]
